# baseline (speedup 1.0000x reference)
import functools

import jax
import jax.numpy as jnp
from jax import lax
from jax.experimental import pallas as pl
from jax.experimental.pallas import tpu as pltpu, tpu_sc as plsc

R = 10000
D = 128
RH = R // 2
Q0 = 2560
RPT = 160
RSTEPS = (160, 152)
RVALID = (Q0, RH - Q0)
TRASH = Q0
ACC = 2576
B = 256
HB = B // 2
NC, NS = 2, 16


def _body(n):
    def body(atoms_hbm, idx_hbm, starts_hbm, ends_hbm, out_hbm,
             featA, featB, idx1dA, idx1dB, iA0, iA1, iB0, iB1,
             lcnt, cbuf2, cbufR, svbuf, evbuf,
             semA, semB, semsA, semsB, accum, cnt_t):
        c = lax.axis_index("c")
        sid = lax.axis_index("s")
        ones16 = jnp.ones((16,), jnp.float32)

        for p in range(2):
            rstep = RSTEPS[p]
            rvalid = RVALID[p]
            base = c * RH + p * Q0
            rb = sid * rstep

            pltpu.sync_copy(starts_hbm.at[c, p, sid], svbuf)
            pltpu.sync_copy(ends_hbm.at[c, p, sid], evbuf)
            start = svbuf[...][0]
            end = evbuf[...][0]

            def zrow(i, _):
                for k in range(D // 16):
                    featA[i, pl.ds(k * 16, 16)] = jnp.zeros((16,),
                                                            jnp.float32)
                return _
            lax.fori_loop(0, RPT, zrow, 0)

            def zcnt(i, _):
                lcnt[pl.ds(i * 16, 16)] = jnp.zeros((16,), jnp.float32)
                return _
            lax.fori_loop(0, ACC // 16, zcnt, 0)

            trash16 = jnp.full((16,), TRASH, jnp.int32)
            for k in range(HB // 16):
                iB0[pl.ds(k * 16, 16)] = trash16
                iB1[pl.ds(k * 16, 16)] = trash16

            pltpu.sync_copy(featA.at[pl.ds(0, RPT)],
                            accum.at[pl.ds(rb, RPT)])
            plsc.subcore_barrier()

            nblk = (end - start + (B - 1)) // B
            npair = jnp.maximum(1, (nblk + 1) // 2)

            def p0_of(bb):
                return pl.multiple_of(
                    jnp.minimum(start + bb * B, n - B), 8)

            def fire_gather(feat, idx1d, sem, bb):
                p0 = p0_of(bb)
                pltpu.async_copy(atoms_hbm.at[pl.ds(p0, B)], feat, sem)
                pltpu.async_copy(idx_hbm.at[pl.ds(p0, B)], idx1d, sem)

            def wait_gather(feat, idx1d, sem):
                pltpu.make_async_copy(
                    atoms_hbm.at[pl.ds(0, B)], feat, sem).wait()
                pltpu.make_async_copy(
                    idx_hbm.at[pl.ds(0, B)], idx1d, sem).wait()

            def remap(idx1d, i0, i1, bb):
                logical = start + bb * B
                p0 = p0_of(bb)
                for k in range(B // 16):
                    v = idx1d[pl.ds(k * 16, 16)]
                    pos = p0 + k * 16 + lax.iota(jnp.int32, 16)
                    local = v - base
                    ok = ((pos >= logical) & (pos < end)
                          & (local >= 0) & (local < rvalid))
                    t = jnp.where(ok, local, TRASH)
                    dst = i0 if k < HB // 16 else i1
                    dst[pl.ds((k % (HB // 16)) * 16, 16)] = t
                    plsc.addupdate_scatter(lcnt, [t], ones16)

            def fire_scat(feat, i0, i1, sem):
                pltpu.async_copy(feat.at[pl.ds(0, HB)],
                                 accum.at[i0], sem, add=True)
                pltpu.async_copy(feat.at[pl.ds(HB, HB)],
                                 accum.at[i1], sem, add=True)

            def wait_scat(feat, i0, i1, sem):
                pltpu.make_async_copy(
                    feat.at[pl.ds(0, HB)], accum.at[i0], sem).wait()
                pltpu.make_async_copy(
                    feat.at[pl.ds(HB, HB)], accum.at[i1], sem).wait()

            fire_scat(featB, iB0, iB1, semsB)
            fire_gather(featA, idx1dA, semA, 0)

            def pair(o, _):
                bb = 2 * o
                wait_gather(featA, idx1dA, semA)
                wait_scat(featB, iB0, iB1, semsB)
                fire_gather(featB, idx1dB, semB, bb + 1)
                remap(idx1dA, iA0, iA1, bb)
                fire_scat(featA, iA0, iA1, semsA)
                wait_gather(featB, idx1dB, semB)
                wait_scat(featA, iA0, iA1, semsA)
                fire_gather(featA, idx1dA, semA, bb + 2)
                remap(idx1dB, iB0, iB1, bb + 1)
                fire_scat(featB, iB0, iB1, semsB)
                return _
            lax.fori_loop(0, npair, pair, 0)
            wait_gather(featA, idx1dA, semA)
            wait_scat(featB, iB0, iB1, semsB)
            pltpu.sync_copy(lcnt, cnt_t.at[sid])
            plsc.subcore_barrier()

            pltpu.sync_copy(accum.at[pl.ds(rb, RPT)],
                            featA.at[pl.ds(0, RPT)])
            pltpu.sync_copy(cnt_t.at[:, pl.ds(rb, RPT)], cbuf2)
            for g in range(RPT // 16):
                tot = cbuf2[0, pl.ds(g * 16, 16)]
                for t in range(1, NS):
                    tot = tot + cbuf2[t, pl.ds(g * 16, 16)]
                cbufR[pl.ds(g * 16, 16)] = jnp.maximum(tot, 1.0)

            def drow(i, _):
                cnt = plsc.load_gather(
                    cbufR, [jnp.full((16,), i, jnp.int32)])
                for k in range(D // 16):
                    featA[i, pl.ds(k * 16, 16)] = (
                        featA[i, pl.ds(k * 16, 16)] / cnt)
                return _
            lax.fori_loop(0, RPT, drow, 0)
            pltpu.sync_copy(featA.at[pl.ds(0, RPT)],
                            out_hbm.at[pl.ds(base + rb, RPT)])
            if p == 0:
                plsc.subcore_barrier()

    return body


@jax.jit
def kernel(atom_features, residue_index):
    n = atom_features.shape[0]

    qb = jnp.array([Q0, RH, RH + Q0], jnp.int32)
    s1, s2, s3 = (jnp.sum(residue_index < q).astype(jnp.int32) for q in qb)
    w = jnp.arange(NS, dtype=jnp.int32)

    def tile_ranges(lo8, hi):
        ln = hi - lo8
        st = lo8 + ((w * ln) // NS) // 8 * 8
        en = jnp.concatenate([st[1:], hi[None]])
        return st, en

    f8 = lambda x: (x // 8) * 8
    r00 = tile_ranges(jnp.int32(0), s1)
    r01 = tile_ranges(f8(s1), s2)
    r10 = tile_ranges(f8(s2), s3)
    r11 = tile_ranges(f8(s3), jnp.int32(n))
    starts = jnp.stack([jnp.stack([r00[0], r01[0]]),
                        jnp.stack([r10[0], r11[0]])])
    ends = jnp.stack([jnp.stack([r00[1], r01[1]]),
                      jnp.stack([r10[1], r11[1]])])
    starts = jnp.broadcast_to(starts[..., None],
                              (NC, 2, NS, 16)).astype(jnp.int32)
    ends = jnp.broadcast_to(ends[..., None],
                            (NC, 2, NS, 16)).astype(jnp.int32)

    mesh = plsc.VectorSubcoreMesh(core_axis_name="c", subcore_axis_name="s")
    out = pl.kernel(
        _body(n),
        out_type=jax.ShapeDtypeStruct((R, D), jnp.float32),
        mesh=mesh,
        compiler_params=pltpu.CompilerParams(
            use_tc_tiling_on_sc=False, needs_layout_passes=False),
        scratch_types=[
            pltpu.VMEM((B, D), jnp.float32),
            pltpu.VMEM((B, D), jnp.float32),
            pltpu.VMEM((B,), jnp.int32),
            pltpu.VMEM((B,), jnp.int32),
            pltpu.VMEM((HB,), jnp.int32),
            pltpu.VMEM((HB,), jnp.int32),
            pltpu.VMEM((HB,), jnp.int32),
            pltpu.VMEM((HB,), jnp.int32),
            pltpu.VMEM((ACC,), jnp.float32),
            pltpu.VMEM((NS, RPT), jnp.float32),
            pltpu.VMEM((RPT,), jnp.float32),
            pltpu.VMEM((16,), jnp.int32),
            pltpu.VMEM((16,), jnp.int32),
            pltpu.SemaphoreType.DMA,
            pltpu.SemaphoreType.DMA,
            pltpu.SemaphoreType.DMA,
            pltpu.SemaphoreType.DMA,
            pltpu.VMEM_SHARED((ACC, D), jnp.float32),
            pltpu.VMEM_SHARED((NS, ACC), jnp.float32),
        ],
    )(atom_features, residue_index, starts, ends)
    return out


# speedup vs baseline: 1.1249x; 1.1249x over previous
import functools

import jax
import jax.numpy as jnp
from jax import lax
from jax.experimental import pallas as pl
from jax.experimental.pallas import tpu as pltpu, tpu_sc as plsc

R = 10000
D = 128
RH = R // 2
RPT = 320
RSTEP = 312
RPAD = 5008
RC = 80
B = 128
NC, NS = 2, 16


def _body(n):
    def body(atoms_hbm, idx_hbm, svec_hbm, out_hbm,
             featA, featB, idxA, idxB, lcnt, cbuf2, cbufR, svbuf,
             semA, semB, semsA, semsB, accum, cnt_t):
        c = lax.axis_index("c")
        sid = lax.axis_index("s")

        pltpu.sync_copy(svec_hbm, svbuf)
        s = svbuf[...][0]
        s8 = (s // 8) * 8
        lo = jnp.where(c == 0, 0, s8)
        hi = jnp.where(c == 0, s, n)
        ln = hi - lo
        start = lo + ((sid * ln) // NS) // 8 * 8
        end = jnp.where(sid == NS - 1,
                        hi, lo + (((sid + 1) * ln) // NS) // 8 * 8)
        base = c * RH

        def zrow(i, _):
            for k in range(D // 16):
                featA[i, pl.ds(k * 16, 16)] = jnp.zeros((16,), jnp.float32)
            idxB[pl.ds((i % 8) * 16, 16)] = jnp.full((16,), RH, jnp.int32)
            return _
        lax.fori_loop(0, B, zrow, 0)

        def zcnt(i, _):
            lcnt[pl.ds(i * 16, 16)] = jnp.zeros((16,), jnp.float32)
            return _
        lax.fori_loop(0, RPAD // 16, zcnt, 0)

        rb = sid * RSTEP
        for j in range(RPT // RC):
            pltpu.sync_copy(featA.at[pl.ds(0, RC)],
                            accum.at[pl.ds(rb + j * RC, RC)])
        plsc.subcore_barrier()

        nblk = (end - start + (B - 1)) // B
        npair = jnp.maximum(1, (nblk + 1) // 2)
        ones16 = jnp.ones((16,), jnp.float32)

        def p0_of(bb):
            return pl.multiple_of(jnp.minimum(start + bb * B, n - B), 8)

        def fire_gather(feat, idx, sem, bb):
            p0 = p0_of(bb)
            pltpu.async_copy(atoms_hbm.at[pl.ds(p0, B)], feat, sem)
            pltpu.async_copy(idx_hbm.at[pl.ds(p0, B)], idx, sem)

        def wait_gather(feat, idx, sem):
            pltpu.make_async_copy(atoms_hbm.at[pl.ds(0, B)], feat, sem).wait()
            pltpu.make_async_copy(idx_hbm.at[pl.ds(0, B)], idx, sem).wait()

        def remap(idx, bb):
            logical = start + bb * B
            p0 = p0_of(bb)
            for k in range(B // 16):
                v = idx[pl.ds(k * 16, 16)]
                pos = p0 + k * 16 + lax.iota(jnp.int32, 16)
                local = v - base
                ok = ((pos >= logical) & (pos < end)
                      & (local >= 0) & (local < RH))
                t = jnp.where(ok, local, RH)
                idx[pl.ds(k * 16, 16)] = t
                plsc.addupdate_scatter(lcnt, [t], ones16)

        def fire_scat(feat, idx, sem):
            pltpu.async_copy(feat, accum.at[idx], sem, add=True)

        def wait_scat(feat, idx, sem):
            pltpu.make_async_copy(feat, accum.at[idx], sem).wait()

        fire_scat(featB, idxB, semsB)
        fire_gather(featA, idxA, semA, 0)

        def pair(o, _):
            bb = 2 * o
            wait_gather(featA, idxA, semA)
            wait_scat(featB, idxB, semsB)
            fire_gather(featB, idxB, semB, bb + 1)
            remap(idxA, bb)
            fire_scat(featA, idxA, semsA)
            wait_gather(featB, idxB, semB)
            wait_scat(featA, idxA, semsA)
            fire_gather(featA, idxA, semA, bb + 2)
            remap(idxB, bb + 1)
            fire_scat(featB, idxB, semsB)
            return _
        lax.fori_loop(0, npair, pair, 0)
        wait_gather(featA, idxA, semA)
        wait_scat(featB, idxB, semsB)
        pltpu.sync_copy(lcnt, cnt_t.at[sid])
        plsc.subcore_barrier()

        for j in range(RPT // RC):
            pltpu.sync_copy(accum.at[pl.ds(rb + j * RC, RC)],
                            featA.at[pl.ds(0, RC)])
            pltpu.sync_copy(cnt_t.at[:, pl.ds(rb + j * RC, RC)], cbuf2)
            for g in range(RC // 16):
                tot = cbuf2[0, pl.ds(g * 16, 16)]
                for t in range(1, NS):
                    tot = tot + cbuf2[t, pl.ds(g * 16, 16)]
                cbufR[pl.ds(g * 16, 16)] = jnp.maximum(tot, 1.0)

            def drow(i, _):
                cnt = plsc.load_gather(cbufR, [jnp.full((16,), i, jnp.int32)])
                for k in range(D // 16):
                    featA[i, pl.ds(k * 16, 16)] = (
                        featA[i, pl.ds(k * 16, 16)] / cnt)
                return _
            lax.fori_loop(0, RC, drow, 0)
            pltpu.sync_copy(featA.at[pl.ds(0, RC)],
                            out_hbm.at[pl.ds(base + rb + j * RC, RC)])

    return body


@jax.jit
def kernel(atom_features, residue_index):
    n = atom_features.shape[0]

    s = jnp.sum(residue_index < RH).astype(jnp.int32)
    svec = jnp.broadcast_to(s[None], (16,)).astype(jnp.int32)

    mesh = plsc.VectorSubcoreMesh(core_axis_name="c", subcore_axis_name="s")
    out = pl.kernel(
        _body(n),
        out_type=jax.ShapeDtypeStruct((R, D), jnp.float32),
        mesh=mesh,
        compiler_params=pltpu.CompilerParams(
            use_tc_tiling_on_sc=False, needs_layout_passes=False),
        scratch_types=[
            pltpu.VMEM((B, D), jnp.float32),
            pltpu.VMEM((B, D), jnp.float32),
            pltpu.VMEM((B,), jnp.int32),
            pltpu.VMEM((B,), jnp.int32),
            pltpu.VMEM((RPAD,), jnp.float32),
            pltpu.VMEM((NS, RC), jnp.float32),
            pltpu.VMEM((RC,), jnp.float32),
            pltpu.VMEM((16,), jnp.int32),
            pltpu.SemaphoreType.DMA,
            pltpu.SemaphoreType.DMA,
            pltpu.SemaphoreType.DMA,
            pltpu.SemaphoreType.DMA,
            pltpu.VMEM_SHARED((RPAD, D), jnp.float32),
            pltpu.VMEM_SHARED((NS, RPAD), jnp.float32),
        ],
    )(atom_features, residue_index, svec)
    return out
